# baseline (speedup 1.0000x reference)
import functools

import jax
import jax.numpy as jnp
from jax import lax
from jax.experimental import pallas as pl
from jax.experimental.pallas import tpu as pltpu
from jax.experimental.pallas import tpu_sc as plsc

NUM_NODES = 1000000
EMB = 64
N_IDX = 819200

NC, NS = 2, 16
NW = NC * NS

_mesh = plsc.VectorSubcoreMesh(core_axis_name="c", subcore_axis_name="s")
_tiled = pltpu.CompilerParams(use_tc_tiling_on_sc=True)
_linear = pltpu.CompilerParams(use_tc_tiling_on_sc=False)

ACHUNK = 256
PCHUNK = ACHUNK // 2
NFULL = NUM_NODES // ACHUNK
ATAIL = NUM_NODES - NFULL * ACHUNK
ABASE = NFULL // NW
AEXTRA = NFULL - ABASE * NW


@functools.partial(
    pl.kernel,
    out_type=jax.ShapeDtypeStruct((NUM_NODES // 2, 128), jnp.float32),
    mesh=_mesh,
    scratch_types=[
        pltpu.VMEM((2, ACHUNK, EMB), jnp.float32),
        pltpu.VMEM((2, PCHUNK, 128), jnp.float32),
        pltpu.SemaphoreType.DMA((2,)),
        pltpu.SemaphoreType.DMA((2,)),
    ],
    compiler_params=_tiled,
)
def _repack_sc(table_hbm, tlin_hbm, buf_v, pak_v, rsem, wsem):
    wid = lax.axis_index("s") * NC + lax.axis_index("c")
    start = wid * ABASE + lax.min(wid, AEXTRA)
    count = ABASE + jnp.where(wid < AEXTRA, 1, 0)

    def fire_read(c, b):
        pltpu.async_copy(
            table_hbm.at[pl.ds(c * ACHUNK, ACHUNK)], buf_v.at[b], rsem.at[b]
        )

    def wait_read(c, b):
        pltpu.make_async_copy(
            table_hbm.at[pl.ds(c * ACHUNK, ACHUNK)], buf_v.at[b], rsem.at[b]
        ).wait()

    def fire_write(c, b):
        pltpu.async_copy(
            pak_v.at[b], tlin_hbm.at[pl.ds(c * PCHUNK, PCHUNK)], wsem.at[b]
        )

    def wait_write(c, b):
        pltpu.make_async_copy(
            pak_v.at[b], tlin_hbm.at[pl.ds(c * PCHUNK, PCHUNK)], wsem.at[b]
        ).wait()

    def pack(b, nrows):
        def prow(r2, carry):
            for u in range(2):
                r = 2 * r2 + u
                for j in range(4):
                    pak_v[b, r, pl.ds(j * 16, 16)] = buf_v[
                        b, 2 * r, pl.ds(j * 16, 16)
                    ]
                    pak_v[b, r, pl.ds(64 + j * 16, 16)] = buf_v[
                        b, 2 * r + 1, pl.ds(j * 16, 16)
                    ]
            return carry

        lax.fori_loop(0, nrows // 2, prow, 0)

    nt = (count + 1) // 2

    def body(t, carry):
        for b in range(2):
            i = 2 * t + b
            c = start + i

            @pl.when(jnp.logical_and(i < count, i >= 2))
            def _():
                wait_write(c - 2, b)

            @pl.when(i < count)
            def _():
                fire_read(c, b)

        for b in range(2):
            i = 2 * t + b
            c = start + i

            @pl.when(i < count)
            def _():
                wait_read(c, b)
                pack(b, PCHUNK)
                fire_write(c, b)

        return carry

    lax.fori_loop(0, nt, body, 0)

    last = count - 1

    @pl.when(count > 1)
    def _():
        wait_write(start + last - 1, (last - 1) % 2)

    @pl.when(count > 0)
    def _():
        wait_write(start + last, last % 2)

    @pl.when(wid == NW - 1)
    def _():
        pltpu.sync_copy(
            table_hbm.at[pl.ds(NFULL * ACHUNK, ATAIL)],
            buf_v.at[0].at[pl.ds(0, ATAIL)],
        )
        pack(0, ATAIL // 2)
        pltpu.sync_copy(
            pak_v.at[0].at[pl.ds(0, ATAIL // 2)],
            tlin_hbm.at[pl.ds(NFULL * PCHUNK, ATAIL // 2)],
        )


KSPLIT = 4
SLOT = 128
ROWS_PER_K = N_IDX // KSPLIT
ROWS_PER_W = ROWS_PER_K // NW
NSLOTS = ROWS_PER_W // SLOT
NGROUPS = N_IDX // SLOT
RING = 5
G = 3
NT = NSLOTS // RING


def _make_gather(k):
    gbase = k * (NGROUPS // KSPLIT)

    @functools.partial(
        pl.kernel,
        out_type=jax.ShapeDtypeStruct((ROWS_PER_K, 128), jnp.float32),
        mesh=_mesh,
        scratch_types=[
            pltpu.VMEM((NSLOTS, SLOT), jnp.int32),
            pltpu.VMEM((RING, SLOT, EMB), jnp.float32),
            pltpu.SemaphoreType.DMA((RING,)),
            pltpu.SemaphoreType.DMA((RING,)),
        ],
        compiler_params=_linear,
        name=f"gather_part{k}",
    )
    def _gather_sc(idx_hbm, tlin_hbm, out_hbm, idx_v, rows_v, gsem, ssem):
        wid = lax.axis_index("s") * NC + lax.axis_index("c")
        row0 = wid * ROWS_PER_W

        pltpu.sync_copy(
            idx_hbm.at[pl.ds(gbase + wid * NSLOTS, NSLOTS)], idx_v
        )

        def fire_gather(s, b):
            pltpu.async_copy(
                tlin_hbm.at[idx_v.at[s]], rows_v.at[b], gsem.at[b]
            )

        def wait_gather(s, b):
            pltpu.make_async_copy(
                tlin_hbm.at[idx_v.at[s]], rows_v.at[b], gsem.at[b]
            ).wait()

        def fire_store(s, b):
            pltpu.async_copy(
                rows_v.at[b],
                out_hbm.at[pl.ds(row0 + s * SLOT, SLOT), pl.ds(0, EMB)],
                ssem.at[b],
            )

        def wait_store(s, b):
            pltpu.make_async_copy(
                rows_v.at[b],
                out_hbm.at[pl.ds(row0 + s * SLOT, SLOT), pl.ds(0, EMB)],
                ssem.at[b],
            ).wait()

        def body(t, carry):
            for b in range(RING):
                s = t * RING + b

                @pl.when(t > 0)
                def _():
                    wait_store(s - RING, b)

                fire_gather(s, b)

                bl = (b - G) % RING

                @pl.when(s >= G)
                def _():
                    wait_gather(s - G, bl)
                    fire_store(s - G, bl)

            return carry

        lax.fori_loop(0, NT, body, 0)

        for j in range(G):
            s = NSLOTS - G + j
            wait_gather(s, s % RING)
            fire_store(s, s % RING)
        for j in range(RING):
            s = NSLOTS - RING + j
            wait_store(s, s % RING)

    return _gather_sc


_gathers = [_make_gather(k) for k in range(KSPLIT)]


BLKC = 2048
NBLK = ROWS_PER_K // BLKC


def _depack_body(i_ref, o_ref):
    o_ref[...] = i_ref[:, :EMB]


def _depack_body_alias(i_ref, prev_ref, o_ref):
    del prev_ref
    o_ref[...] = i_ref[:, :EMB]


def _make_depack(k):
    common = dict(
        grid=(NBLK,),
        out_shape=jax.ShapeDtypeStruct((N_IDX, EMB), jnp.float32),
    )
    in_spec = pl.BlockSpec((BLKC, 128), lambda i: (i, 0))
    out_spec = pl.BlockSpec((BLKC, EMB), lambda i, _k=k: (_k * NBLK + i, 0))
    if k == 0:
        return pl.pallas_call(
            _depack_body,
            in_specs=[in_spec],
            out_specs=out_spec,
            **common,
        )
    return pl.pallas_call(
        _depack_body_alias,
        in_specs=[in_spec, pl.BlockSpec(memory_space=pl.ANY)],
        out_specs=out_spec,
        input_output_aliases={1: 0},
        **common,
    )


_depacks = [_make_depack(k) for k in range(KSPLIT)]


def kernel(node_idx, node_embs):
    tlin = _repack_sc(node_embs)
    tview = tlin.reshape(NUM_NODES, EMB)
    idx2d = node_idx.reshape(NGROUPS, SLOT)
    out = None
    for k in range(KSPLIT):
        part = _gathers[k](idx2d, tview)
        if k == 0:
            out = _depacks[0](part)
        else:
            out = _depacks[k](part, out)
    return out


# speedup vs baseline: 1.2289x; 1.2289x over previous
import functools

import jax
import jax.numpy as jnp
from jax import lax
from jax.experimental import pallas as pl
from jax.experimental.pallas import tpu as pltpu
from jax.experimental.pallas import tpu_sc as plsc

NUM_NODES = 1000000
EMB = 64
N_IDX = 819200

NC, NS = 2, 16
NW = NC * NS

SLOT = 128
ROWS_PER_W = N_IDX // NW
NSLOTS = ROWS_PER_W // SLOT
NGROUPS = N_IDX // SLOT
RING = 5
G = 3
NT = NSLOTS // RING

_mesh = plsc.VectorSubcoreMesh(core_axis_name="c", subcore_axis_name="s")
_linear = pltpu.CompilerParams(use_tc_tiling_on_sc=False)


@functools.partial(
    pl.kernel,
    out_type=jax.ShapeDtypeStruct((N_IDX, 128), jnp.float32),
    mesh=_mesh,
    scratch_types=[
        pltpu.VMEM((NSLOTS, SLOT), jnp.int32),
        pltpu.VMEM((RING, SLOT, EMB), jnp.float32),
        pltpu.SemaphoreType.DMA((RING,)),
        pltpu.SemaphoreType.DMA((RING,)),
    ],
    compiler_params=_linear,
    name="gather_rows",
)
def _gather_sc(idx_hbm, tlin_hbm, out_hbm, idx_v, rows_v, gsem, ssem):
    wid = lax.axis_index("s") * NC + lax.axis_index("c")
    row0 = wid * ROWS_PER_W

    pltpu.sync_copy(idx_hbm.at[pl.ds(wid * NSLOTS, NSLOTS)], idx_v)

    def fire_gather(s, b):
        pltpu.async_copy(tlin_hbm.at[idx_v.at[s]], rows_v.at[b], gsem.at[b])

    def wait_gather(s, b):
        pltpu.make_async_copy(
            tlin_hbm.at[idx_v.at[s]], rows_v.at[b], gsem.at[b]
        ).wait()

    def fire_store(s, b):
        pltpu.async_copy(
            rows_v.at[b],
            out_hbm.at[pl.ds(row0 + s * SLOT, SLOT), pl.ds(0, EMB)],
            ssem.at[b],
        )

    def wait_store(s, b):
        pltpu.make_async_copy(
            rows_v.at[b],
            out_hbm.at[pl.ds(row0 + s * SLOT, SLOT), pl.ds(0, EMB)],
            ssem.at[b],
        ).wait()

    def body(t, carry):
        for b in range(RING):
            s = t * RING + b

            @pl.when(t > 0)
            def _():
                wait_store(s - RING, b)

            fire_gather(s, b)

            bl = (b - G) % RING

            @pl.when(s >= G)
            def _():
                wait_gather(s - G, bl)
                fire_store(s - G, bl)

        return carry

    lax.fori_loop(0, NT, body, 0)

    for j in range(G):
        s = NSLOTS - G + j
        wait_gather(s, s % RING)
        fire_store(s, s % RING)
    for j in range(RING):
        s = NSLOTS - RING + j
        wait_store(s, s % RING)


BLKC = 2048
NBLK = N_IDX // BLKC


def _depack_body(i_ref, o_ref):
    o_ref[...] = i_ref[:, :EMB]


_depack = pl.pallas_call(
    _depack_body,
    grid=(NBLK,),
    in_specs=[pl.BlockSpec((BLKC, 128), lambda i: (i, 0))],
    out_specs=pl.BlockSpec((BLKC, EMB), lambda i: (i, 0)),
    out_shape=jax.ShapeDtypeStruct((N_IDX, EMB), jnp.float32),
)


def kernel(node_idx, node_embs):
    t2 = node_embs.reshape(NUM_NODES // 2, 128)
    tview = t2.reshape(NUM_NODES, EMB)
    idx2d = node_idx.reshape(NGROUPS, SLOT)
    part = _gather_sc(idx2d, tview)
    return _depack(part)


# speedup vs baseline: 1.5521x; 1.2630x over previous
import functools

import jax
import jax.numpy as jnp
from jax import lax
from jax.experimental import pallas as pl
from jax.experimental.pallas import tpu as pltpu
from jax.experimental.pallas import tpu_sc as plsc

NUM_NODES = 1000000
EMB = 64
N_IDX = 819200

NC, NS = 2, 16
NW = NC * NS

SLOT = 128
ROWS_PER_W = N_IDX // NW
NSLOTS = ROWS_PER_W // SLOT
NGROUPS = N_IDX // SLOT
RING = 5
G = 3
NT = NSLOTS // RING

_mesh = plsc.VectorSubcoreMesh(core_axis_name="c", subcore_axis_name="s")
_linear = pltpu.CompilerParams(use_tc_tiling_on_sc=False)


@functools.partial(
    pl.kernel,
    out_type=jax.ShapeDtypeStruct((N_IDX, 128), jnp.float32),
    mesh=_mesh,
    scratch_types=[
        pltpu.VMEM((NSLOTS, SLOT), jnp.int32),
        pltpu.VMEM((RING, SLOT, EMB), jnp.float32),
        pltpu.SemaphoreType.DMA((RING,)),
        pltpu.SemaphoreType.DMA((RING,)),
    ],
    compiler_params=_linear,
    name="gather_rows",
)
def _gather_sc(idx_hbm, tlin_hbm, out_hbm, idx_v, rows_v, gsem, ssem):
    wid = lax.axis_index("s") * NC + lax.axis_index("c")
    row0 = wid * ROWS_PER_W

    pltpu.sync_copy(idx_hbm.at[pl.ds(wid * NSLOTS, NSLOTS)], idx_v)

    def fire_gather(s, b):
        pltpu.async_copy(tlin_hbm.at[idx_v.at[s]], rows_v.at[b], gsem.at[b])

    def wait_gather(s, b):
        pltpu.make_async_copy(
            tlin_hbm.at[idx_v.at[s]], rows_v.at[b], gsem.at[b]
        ).wait()

    def fire_store(s, b):
        pltpu.async_copy(
            rows_v.at[b],
            out_hbm.at[pl.ds(row0 + s * SLOT, SLOT), pl.ds(0, EMB)],
            ssem.at[b],
        )

    def wait_store(s, b):
        pltpu.make_async_copy(
            rows_v.at[b],
            out_hbm.at[pl.ds(row0 + s * SLOT, SLOT), pl.ds(0, EMB)],
            ssem.at[b],
        ).wait()

    def body(t, carry):
        for b in range(RING):
            s = t * RING + b

            @pl.when(t > 0)
            def _():
                wait_store(s - RING, b)

            fire_gather(s, b)

            bl = (b - G) % RING

            @pl.when(s >= G)
            def _():
                wait_gather(s - G, bl)
                fire_store(s - G, bl)

        return carry

    lax.fori_loop(0, NT, body, 0)

    for j in range(G):
        s = NSLOTS - G + j
        wait_gather(s, s % RING)
        fire_store(s, s % RING)
    for j in range(RING):
        s = NSLOTS - RING + j
        wait_store(s, s % RING)


BLKC = 2048
NBLK = N_IDX // BLKC


def _depack_body(i_ref, o_ref):
    o_ref[...] = i_ref[:, :EMB].T


_depack = pl.pallas_call(
    _depack_body,
    grid=(NBLK,),
    in_specs=[pl.BlockSpec((BLKC, 128), lambda i: (i, 0))],
    out_specs=pl.BlockSpec((EMB, BLKC), lambda i: (0, i)),
    out_shape=jax.ShapeDtypeStruct((EMB, N_IDX), jnp.float32),
)


def kernel(node_idx, node_embs):
    t2 = node_embs.reshape(NUM_NODES // 2, 128)
    tview = t2.reshape(NUM_NODES, EMB)
    idx2d = node_idx.reshape(NGROUPS, SLOT)
    part = _gather_sc(idx2d, tview)
    return _depack(part).T


# speedup vs baseline: 1.7568x; 1.1319x over previous
import functools

import jax
import jax.numpy as jnp
from jax import lax
from jax.experimental import pallas as pl
from jax.experimental.pallas import tpu as pltpu
from jax.experimental.pallas import tpu_sc as plsc

NUM_NODES = 1000000
EMB = 64
N_IDX = 819200

NC, NS = 2, 16
NW = NC * NS

SLOT = 128
ROWS_PER_W = N_IDX // NW
NSLOTS = ROWS_PER_W // SLOT
NGROUPS = N_IDX // SLOT
RING = 5
G = 3
NT = NSLOTS // RING

_mesh = plsc.VectorSubcoreMesh(core_axis_name="c", subcore_axis_name="s")
_linear = pltpu.CompilerParams(use_tc_tiling_on_sc=False)


@functools.partial(
    pl.kernel,
    out_type=jax.ShapeDtypeStruct((N_IDX, 128), jnp.float32),
    mesh=_mesh,
    scratch_types=[
        pltpu.VMEM((NSLOTS, SLOT), jnp.int32),
        pltpu.VMEM((RING, SLOT, EMB), jnp.float32),
        pltpu.SemaphoreType.DMA((RING,)),
        pltpu.SemaphoreType.DMA((RING,)),
    ],
    compiler_params=_linear,
    name="gather_rows",
)
def _gather_sc(idx_hbm, tlin_hbm, out_hbm, idx_v, rows_v, gsem, ssem):
    wid = lax.axis_index("s") * NC + lax.axis_index("c")
    row0 = wid * ROWS_PER_W

    pltpu.sync_copy(idx_hbm.at[pl.ds(wid * NSLOTS, NSLOTS)], idx_v)

    def fire_gather(s, b):
        pltpu.async_copy(tlin_hbm.at[idx_v.at[s]], rows_v.at[b], gsem.at[b])

    def wait_gather(s, b):
        pltpu.make_async_copy(
            tlin_hbm.at[idx_v.at[s]], rows_v.at[b], gsem.at[b]
        ).wait()

    def fire_store(s, b):
        pltpu.async_copy(
            rows_v.at[b],
            out_hbm.at[pl.ds(row0 + s * SLOT, SLOT), pl.ds(0, EMB)],
            ssem.at[b],
        )

    def wait_store(s, b):
        pltpu.make_async_copy(
            rows_v.at[b],
            out_hbm.at[pl.ds(row0 + s * SLOT, SLOT), pl.ds(0, EMB)],
            ssem.at[b],
        ).wait()

    def body(t, carry):
        for b in range(RING):
            s = t * RING + b

            @pl.when(t > 0)
            def _():
                wait_store(s - RING, b)

            fire_gather(s, b)

            bl = (b - G) % RING

            @pl.when(s >= G)
            def _():
                wait_gather(s - G, bl)
                fire_store(s - G, bl)

        return carry

    lax.fori_loop(0, NT, body, 0)

    for j in range(G):
        s = NSLOTS - G + j
        wait_gather(s, s % RING)
        fire_store(s, s % RING)
    for j in range(RING):
        s = NSLOTS - RING + j
        wait_store(s, s % RING)


BLKT = 2048
NBLKT = -(-NUM_NODES // BLKT)


def _tpose_body(i_ref, o_ref):
    o_ref[:, :EMB] = i_ref[...].T


_tpose = pl.pallas_call(
    _tpose_body,
    grid=(NBLKT,),
    in_specs=[pl.BlockSpec((EMB, BLKT), lambda i: (0, i))],
    out_specs=pl.BlockSpec((BLKT, 128), lambda i: (i, 0)),
    out_shape=jax.ShapeDtypeStruct((NUM_NODES, 128), jnp.float32),
)


BLKC = 2048
NBLK = N_IDX // BLKC


def _depack_body(i_ref, o_ref):
    o_ref[...] = i_ref[:, :EMB].T


_depack = pl.pallas_call(
    _depack_body,
    grid=(NBLK,),
    in_specs=[pl.BlockSpec((BLKC, 128), lambda i: (i, 0))],
    out_specs=pl.BlockSpec((EMB, BLKC), lambda i: (0, i)),
    out_shape=jax.ShapeDtypeStruct((EMB, N_IDX), jnp.float32),
)


def kernel(node_idx, node_embs):
    tpad = _tpose(node_embs.T)
    tview = tpad.reshape(2 * NUM_NODES, EMB)
    idx2d = (node_idx * 2).reshape(NGROUPS, SLOT)
    part = _gather_sc(idx2d, tview)
    return _depack(part).T


# speedup vs baseline: 1.8183x; 1.0350x over previous
import functools

import jax
import jax.numpy as jnp
from jax import lax
from jax.experimental import pallas as pl
from jax.experimental.pallas import tpu as pltpu
from jax.experimental.pallas import tpu_sc as plsc

NUM_NODES = 1000000
EMB = 64
N_IDX = 819200

NC, NS = 2, 16
NW = NC * NS

KSPLIT = 4
SLOT = 128
ROWS_PER_K = N_IDX // KSPLIT
ROWS_PER_W = ROWS_PER_K // NW
NSLOTS = ROWS_PER_W // SLOT
NGROUPS = N_IDX // SLOT
RING = 5
G = 3
NT = NSLOTS // RING

_mesh = plsc.VectorSubcoreMesh(core_axis_name="c", subcore_axis_name="s")
_linear = pltpu.CompilerParams(use_tc_tiling_on_sc=False)


def _make_gather(k):
    gbase = k * (NGROUPS // KSPLIT)

    @functools.partial(
        pl.kernel,
        out_type=jax.ShapeDtypeStruct((ROWS_PER_K, 128), jnp.float32),
        mesh=_mesh,
        scratch_types=[
            pltpu.VMEM((NSLOTS, SLOT), jnp.int32),
            pltpu.VMEM((RING, SLOT, EMB), jnp.float32),
            pltpu.SemaphoreType.DMA((RING,)),
            pltpu.SemaphoreType.DMA((RING,)),
        ],
        compiler_params=_linear,
        name=f"gather_rows{k}",
    )
    def _gather_sc(idx_hbm, tlin_hbm, out_hbm, idx_v, rows_v, gsem, ssem):
        wid = lax.axis_index("s") * NC + lax.axis_index("c")
        row0 = wid * ROWS_PER_W

        pltpu.sync_copy(idx_hbm.at[pl.ds(gbase + wid * NSLOTS, NSLOTS)], idx_v)

        def fire_gather(s, b):
            pltpu.async_copy(tlin_hbm.at[idx_v.at[s]], rows_v.at[b], gsem.at[b])

        def wait_gather(s, b):
            pltpu.make_async_copy(
                tlin_hbm.at[idx_v.at[s]], rows_v.at[b], gsem.at[b]
            ).wait()

        def fire_store(s, b):
            pltpu.async_copy(
                rows_v.at[b],
                out_hbm.at[pl.ds(row0 + s * SLOT, SLOT), pl.ds(0, EMB)],
                ssem.at[b],
            )

        def wait_store(s, b):
            pltpu.make_async_copy(
                rows_v.at[b],
                out_hbm.at[pl.ds(row0 + s * SLOT, SLOT), pl.ds(0, EMB)],
                ssem.at[b],
            ).wait()

        def body(t, carry):
            for b in range(RING):
                s = t * RING + b

                @pl.when(t > 0)
                def _():
                    wait_store(s - RING, b)

                fire_gather(s, b)

                bl = (b - G) % RING

                @pl.when(s >= G)
                def _():
                    wait_gather(s - G, bl)
                    fire_store(s - G, bl)

            return carry

        lax.fori_loop(0, NT, body, 0)

        for j in range(G):
            s = NSLOTS - G + j
            wait_gather(s, s % RING)
            fire_store(s, s % RING)
        for j in range(RING):
            s = NSLOTS - RING + j
            wait_store(s, s % RING)

    return _gather_sc


_gathers = [_make_gather(k) for k in range(KSPLIT)]


BLKT = 2048
NBLKT = -(-NUM_NODES // BLKT)


def _tpose_body(i_ref, o_ref):
    o_ref[:, :EMB] = i_ref[...].T


_tpose = pl.pallas_call(
    _tpose_body,
    grid=(NBLKT,),
    in_specs=[pl.BlockSpec((EMB, BLKT), lambda i: (0, i))],
    out_specs=pl.BlockSpec((BLKT, 128), lambda i: (i, 0)),
    out_shape=jax.ShapeDtypeStruct((NUM_NODES, 128), jnp.float32),
)


BLKC = 2048
NBLK = ROWS_PER_K // BLKC


def _depack_body(i_ref, o_ref):
    o_ref[...] = i_ref[:, :EMB].T


def _depack_body_alias(i_ref, prev_ref, o_ref):
    del prev_ref
    o_ref[...] = i_ref[:, :EMB].T


def _make_depack(k):
    common = dict(
        grid=(NBLK,),
        out_shape=jax.ShapeDtypeStruct((EMB, N_IDX), jnp.float32),
    )
    in_spec = pl.BlockSpec((BLKC, 128), lambda i: (i, 0))
    out_spec = pl.BlockSpec((EMB, BLKC), lambda i, _k=k: (0, _k * NBLK + i))
    if k == 0:
        return pl.pallas_call(
            _depack_body, in_specs=[in_spec], out_specs=out_spec, **common
        )
    return pl.pallas_call(
        _depack_body_alias,
        in_specs=[in_spec, pl.BlockSpec(memory_space=pl.ANY)],
        out_specs=out_spec,
        input_output_aliases={1: 0},
        **common,
    )


_depacks = [_make_depack(k) for k in range(KSPLIT)]


def kernel(node_idx, node_embs):
    tpad = _tpose(node_embs.T)
    tview = tpad.reshape(2 * NUM_NODES, EMB)
    idx2d = (node_idx * 2).reshape(NGROUPS, SLOT)
    out_t = None
    for k in range(KSPLIT):
        part = _gathers[k](idx2d, tview)
        if k == 0:
            out_t = _depacks[0](part)
        else:
            out_t = _depacks[k](part, out_t)
    return out_t.T
